# baseline (speedup 1.0000x reference)
import functools

import jax
import jax.numpy as jnp
import numpy as np
from jax import lax
from jax.experimental import pallas as pl
from jax.experimental.pallas import tpu as pltpu
from jax.experimental.pallas import tpu_sc as plsc

BATCH = 16384
HIST = 200
EMBED = 64
NCLS = 100
VOCAB = 1000000

NC = 2
NS = 16
NW = NC * NS
S_PER_W = BATCH // NW
SB = 64
NBLK = S_PER_W // SB
G = 8
NG = SB // G
CH0 = 104
CH1 = HIST - CH0
RU = 8
LANES = 16
PK = EMBED // (2 * LANES)

R_W = VOCAB // NW
CC = 250
NCH = R_W // CC

_PERM = np.concatenate([
    np.arange(0, 16), np.arange(32, 48),
    np.arange(16, 32), np.arange(48, 64)])

_mesh = plsc.VectorSubcoreMesh(core_axis_name="c", subcore_axis_name="s")
_cparams = pltpu.CompilerParams(use_tc_tiling_on_sc=False)


@functools.partial(
    pl.kernel,
    mesh=_mesh,
    out_type=jax.ShapeDtypeStruct((VOCAB, 2 * LANES), jnp.int32),
    scratch_types=[
        pltpu.VMEM((2, CC, EMBED), jnp.float32),
        pltpu.VMEM((2, CC, 2 * LANES), jnp.int32),
        pltpu.SemaphoreType.DMA,
        pltpu.SemaphoreType.DMA,
        pltpu.SemaphoreType.DMA,
        pltpu.SemaphoreType.DMA,
    ],
    compiler_params=_cparams,
)
def _convert_kernel(table_hbm, packed_hbm, in_v, out_v,
                    isem_a, isem_b, osem_a, osem_b):
    wid = lax.axis_index("s") * NC + lax.axis_index("c")
    base = wid * R_W
    isems = (isem_a, isem_b)
    osems = (osem_a, osem_b)

    def start_in(k, buf):
        pltpu.async_copy(
            table_hbm.at[pl.ds(base + k * CC, CC)], in_v.at[buf], isems[buf])

    def wait_in(buf):
        pltpu.make_async_copy(
            table_hbm.at[pl.ds(0, CC)], in_v.at[buf], isems[buf]).wait()

    def start_out(k, buf):
        pltpu.async_copy(
            out_v.at[buf], packed_hbm.at[pl.ds(base + k * CC, CC)],
            osems[buf])

    def drain_out(buf):
        pltpu.make_async_copy(
            out_v.at[buf], packed_hbm.at[pl.ds(0, CC)], osems[buf]).wait()

    def convert(buf):
        one = jnp.int32(1)
        rne = jnp.int32(0x7FFF)

        def bf16_bits(v):
            u = lax.bitcast_convert_type(v, jnp.int32)
            r = jnp.bitwise_and(lax.shift_right_logical(u, 16), one)
            return lax.shift_right_logical(u + rne + r, 16)

        def body(r, c):
            for half in range(2):
                a = in_v[buf, r, pl.ds(half * 2 * LANES, LANES)]
                b = in_v[buf, r, pl.ds(half * 2 * LANES + LANES, LANES)]
                out_v[buf, r, pl.ds(half * LANES, LANES)] = (
                    jnp.bitwise_or(bf16_bits(a),
                                   lax.shift_left(bf16_bits(b), 16)))
            return c
        lax.fori_loop(0, CC, body, 0)

    def chunk(k, buf):
        wait_in(buf)

        @pl.when(k > 1)
        def _():
            drain_out(buf)

        convert(buf)
        start_out(k, buf)

        @pl.when(k + 2 < NCH)
        def _():
            start_in(k + 2, buf)

    start_in(0, 0)
    start_in(1, 1)

    def pair(p, c):
        chunk(2 * p, 0)
        chunk(2 * p + 1, 1)
        return c

    lax.fori_loop(0, (NCH - 1) // 2, pair, 0)
    chunk(NCH - 1, 0)
    drain_out(1)
    drain_out(0)


@functools.partial(
    pl.kernel,
    mesh=_mesh,
    out_type=jax.ShapeDtypeStruct((BATCH, EMBED), jnp.float32),
    scratch_types=[
        pltpu.VMEM((SB * HIST,), jnp.int32),
        pltpu.VMEM((2, G * HIST, 2 * LANES), jnp.int32),
        pltpu.VMEM((SB, EMBED), jnp.float32),
        pltpu.SemaphoreType.DMA,
        pltpu.SemaphoreType.DMA,
    ],
    compiler_params=_cparams,
)
def _pool_kernel(x_hbm, table_hbm, pooled_hbm, idx_v, rows_v, pooled_v,
                 sem_a, sem_b):
    wid = lax.axis_index("s") * NC + lax.axis_index("c")
    base = wid * S_PER_W
    himask = jnp.int32(-65536)

    def fire_group(g, buf, sem):
        for j in range(G):
            s_local = g * G + j
            off = pl.multiple_of(s_local * HIST, 8)
            pltpu.async_copy(
                table_hbm.at[idx_v.at[pl.ds(off, CH0)]],
                rows_v.at[buf, pl.ds(j * HIST, CH0)], sem)
            off1 = pl.multiple_of(s_local * HIST + CH0, 8)
            pltpu.async_copy(
                table_hbm.at[idx_v.at[pl.ds(off1, CH1)]],
                rows_v.at[buf, pl.ds(j * HIST + CH0, CH1)], sem)

    def drain_group(buf, sem):
        pltpu.make_async_copy(
            table_hbm.at[pl.ds(0, G * HIST)], rows_v.at[buf], sem).wait()

    def accumulate(buf, g):
        for j in range(G):
            base_row = j * HIST
            zero = jnp.zeros((LANES,), jnp.float32)

            def body(r, acc):
                acc = list(acc)
                for rr in range(RU):
                    row = base_row + r * RU + rr
                    st = (rr % 2) * 4
                    for q in range(PK):
                        w = rows_v[buf, row, pl.ds(q * LANES, LANES)]
                        lo = lax.bitcast_convert_type(
                            lax.shift_left(w, 16), jnp.float32)
                        hi = lax.bitcast_convert_type(
                            jnp.bitwise_and(w, himask), jnp.float32)
                        acc[st + q] = acc[st + q] + lo
                        acc[st + 2 + q] = acc[st + 2 + q] + hi
                return tuple(acc)

            acc = lax.fori_loop(0, HIST // RU, body, (zero,) * 8)
            s_local = g * G + j
            for h in range(4):
                pooled_v[s_local, pl.ds(h * LANES, LANES)] = (
                    acc[h] + acc[4 + h])

    def block_body(blk, carry):
        row0 = base + blk * SB
        pltpu.sync_copy(x_hbm.at[pl.ds(row0 * HIST, SB * HIST)], idx_v)
        fire_group(0, 0, sem_a)

        def two_groups(p, c):
            g0 = 2 * p
            fire_group(g0 + 1, 1, sem_b)
            drain_group(0, sem_a)
            accumulate(0, g0)

            @pl.when(g0 + 2 < NG)
            def _():
                fire_group(g0 + 2, 0, sem_a)

            drain_group(1, sem_b)
            accumulate(1, g0 + 1)
            return c

        lax.fori_loop(0, NG // 2, two_groups, 0)
        pltpu.sync_copy(pooled_v, pooled_hbm.at[pl.ds(row0, SB)])
        return carry

    lax.fori_loop(0, NBLK, block_body, 0)


def _cls_body(p_ref, w_ref, b_ref, o_ref):
    o_ref[...] = lax.dot_general(
        p_ref[...], w_ref[...], (((1,), (1,)), ((), ())),
        preferred_element_type=jnp.float32) * (1.0 / HIST) + b_ref[...]


_BM = 2048


def kernel(x, table, W, b):
    x_flat = x.astype(jnp.int32).reshape(BATCH * HIST)
    packed = _convert_kernel(table)
    pooled = _pool_kernel(x_flat, packed)
    w_perm = W[:, _PERM]
    out = pl.pallas_call(
        _cls_body,
        grid=(BATCH // _BM,),
        in_specs=[
            pl.BlockSpec((_BM, EMBED), lambda i: (i, 0)),
            pl.BlockSpec((NCLS, EMBED), lambda i: (0, 0)),
            pl.BlockSpec((1, NCLS), lambda i: (0, 0)),
        ],
        out_specs=pl.BlockSpec((_BM, NCLS), lambda i: (i, 0)),
        out_shape=jax.ShapeDtypeStruct((BATCH, NCLS), jnp.float32),
    )(pooled, w_perm, b.reshape(1, NCLS))
    return out


# speedup vs baseline: 1.4222x; 1.4222x over previous
import functools

import jax
import jax.numpy as jnp
from jax import lax
from jax.experimental import pallas as pl
from jax.experimental.pallas import tpu as pltpu
from jax.experimental.pallas import tpu_sc as plsc

BATCH = 16384
HIST = 200
EMBED = 64
NCLS = 100

NC = 2
NS = 16
NW = NC * NS
S_PER_W = BATCH // NW
SB = 64
NBLK = S_PER_W // SB
NBUF = 8
LOOK = 6
CH0 = 104
CH1 = HIST - CH0
RU = 8
LANES = 16
NSEG = EMBED // LANES

_mesh = plsc.VectorSubcoreMesh(core_axis_name="c", subcore_axis_name="s")


@functools.partial(
    pl.kernel,
    mesh=_mesh,
    out_type=jax.ShapeDtypeStruct((BATCH, EMBED), jnp.float32),
    scratch_types=[
        pltpu.VMEM((SB * HIST,), jnp.int32),
        pltpu.VMEM((NBUF, HIST, EMBED), jnp.float32),
        pltpu.VMEM((SB, EMBED), jnp.float32),
        [pltpu.SemaphoreType.DMA] * NBUF,
    ],
    compiler_params=pltpu.CompilerParams(use_tc_tiling_on_sc=False),
)
def _pool_kernel(x_hbm, table_hbm, pooled_hbm, idx_v, rows_v, pooled_v, sems):
    wid = lax.axis_index("s") * NC + lax.axis_index("c")
    base = wid * S_PER_W

    def fire(s_local, buf):
        off = pl.multiple_of(s_local * HIST, 8)
        pltpu.async_copy(
            table_hbm.at[idx_v.at[pl.ds(off, CH0)]],
            rows_v.at[buf, pl.ds(0, CH0)], sems[buf])
        off1 = pl.multiple_of(s_local * HIST + CH0, 8)
        pltpu.async_copy(
            table_hbm.at[idx_v.at[pl.ds(off1, CH1)]],
            rows_v.at[buf, pl.ds(CH0, CH1)], sems[buf])

    def drain(buf):
        pltpu.make_async_copy(
            table_hbm.at[pl.ds(0, HIST)], rows_v.at[buf], sems[buf]).wait()

    def accumulate(buf, s_local):
        zero = jnp.zeros((LANES,), jnp.float32)

        def body(r, acc):
            acc = list(acc)
            for rr in range(RU):
                row = r * RU + rr
                half = (rr % 2) * NSEG
                for d in range(NSEG):
                    acc[half + d] = acc[half + d] + rows_v[
                        buf, row, pl.ds(d * LANES, LANES)]
            return tuple(acc)

        acc = lax.fori_loop(0, HIST // RU, body, (zero,) * (2 * NSEG))
        for d in range(NSEG):
            pooled_v[s_local, pl.ds(d * LANES, LANES)] = acc[d] + acc[NSEG + d]

    def block_body(blk, carry):
        row0 = base + blk * SB
        pltpu.sync_copy(x_hbm.at[pl.ds(row0 * HIST, SB * HIST)], idx_v)
        for u in range(LOOK):
            fire(u, u)

        def octet(it, c):
            s0 = it * NBUF
            for u in range(NBUF):
                s = s0 + u
                drain(u)

                @pl.when(s + LOOK < SB)
                def _():
                    fire(s + LOOK, (u + LOOK) % NBUF)

                accumulate(u, s)
            return c

        lax.fori_loop(0, SB // NBUF, octet, 0)
        pltpu.sync_copy(pooled_v, pooled_hbm.at[pl.ds(row0, SB)])
        return carry

    lax.fori_loop(0, NBLK, block_body, 0)


def _cls_body(p_ref, w_ref, b_ref, o_ref):
    o_ref[...] = lax.dot_general(
        p_ref[...], w_ref[...], (((1,), (1,)), ((), ())),
        preferred_element_type=jnp.float32) * (1.0 / HIST) + b_ref[...]


_BM = 2048


def kernel(x, table, W, b):
    x_flat = x.astype(jnp.int32).reshape(BATCH * HIST)
    pooled = _pool_kernel(x_flat, table)
    out = pl.pallas_call(
        _cls_body,
        grid=(BATCH // _BM,),
        in_specs=[
            pl.BlockSpec((_BM, EMBED), lambda i: (i, 0)),
            pl.BlockSpec((NCLS, EMBED), lambda i: (0, 0)),
            pl.BlockSpec((1, NCLS), lambda i: (0, 0)),
        ],
        out_specs=pl.BlockSpec((_BM, NCLS), lambda i: (i, 0)),
        out_shape=jax.ShapeDtypeStruct((BATCH, NCLS), jnp.float32),
    )(pooled, W, b.reshape(1, NCLS))
    return out
